# baseline (speedup 1.0000x reference)
import dataclasses
import functools

import jax
import jax.numpy as jnp
from jax import lax
from jax.experimental import pallas as pl
from jax.experimental.pallas import tpu as pltpu
from jax.experimental.pallas import tpu_sc as plsc

_B, _NBOX, _NNEG = 4, 50, 100
_H, _W = 248, 216
_HW = _H * _W
_PSTR = 6 * _HW
_NPOS = _B * _NBOX
_NBG = _B * _NNEG
_NPOS_PAD = 208
_NIDX = 2 * _NPOS_PAD + _NBG
_NVAL = 3 * _NPOS_PAD + _NBG
_OFF_RT = 0
_OFF_CT = 400
_OFF_GT = 1600
_OFF_INV = 2400
_NPACK = 2416
_ALPHA = 0.25
_BETA_LOC = 2.0
_LN2 = 0.6931471805599453


def _ln(p):
    bits = lax.bitcast_convert_type(p, jnp.int32)
    e = jnp.right_shift(bits, 23) - 127
    m = lax.bitcast_convert_type(
        jnp.bitwise_or(jnp.bitwise_and(bits, 0x007FFFFF), 0x3F800000),
        jnp.float32)
    t = (m - 1.0) / (m + 1.0)
    t2 = t * t
    ln_m = t * (2.0 + t2 * (2.0 / 3.0 + t2 * (2.0 / 5.0
                + t2 * (2.0 / 7.0 + t2 * (2.0 / 9.0)))))
    return e.astype(jnp.float32) * _LN2 + ln_m


def _focal(p):
    one_m = 1.0 - p
    return -_ln(p) * (_ALPHA * one_m * one_m)


def _huber(x):
    ax = jnp.abs(x)
    return jnp.where(ax < 1.0, 0.5 * x * x, ax - 0.5)


_mesh = plsc.VectorSubcoreMesh(core_axis_name="c", subcore_axis_name="s")

_cp = pltpu.CompilerParams()
if "needs_layout_passes" in pltpu.CompilerParams.__dataclass_fields__:
    _cp = dataclasses.replace(_cp, needs_layout_passes=False)


@functools.partial(
    pl.kernel,
    out_type=jax.ShapeDtypeStruct((16,), jnp.float32),
    mesh=_mesh,
    compiler_params=_cp,
    scratch_types=[
        pltpu.VMEM((_NPACK,), jnp.int32),
        pltpu.VMEM((_NIDX,), jnp.int32),
        pltpu.VMEM((_NVAL,), jnp.float32),
        pltpu.VMEM((16,), jnp.float32),
        pltpu.SemaphoreType.DMA,
    ],
)
def _loss_kernel(pk_hbm, loc_hbm, clf_hbm, out_hbm,
                 pk_v, idx_v, val_v, out_v, sem):
    cid = lax.axis_index("c")
    sid = lax.axis_index("s")

    @pl.when(jnp.logical_and(cid == 0, sid == 0))
    def _():
        pltpu.sync_copy(pk_hbm, pk_v)

        lanes = lax.iota(jnp.int32, 16)

        for i in range(_NPOS_PAD // 16):
            p = lanes + (i * 16)
            valid = p < _NPOS
            psafe = jnp.where(valid, p, 0)
            x = plsc.load_gather(pk_v, [psafe * 2])
            y = plsc.load_gather(pk_v, [psafe * 2 + 1])
            b = (jnp.where(p >= _NBOX, 1, 0)
                 + jnp.where(p >= 2 * _NBOX, 1, 0)
                 + jnp.where(p >= 3 * _NBOX, 1, 0))
            base = jnp.where(valid, b * _PSTR + y * _W + x, 0)
            idx_v[pl.ds(i * 16, 16)] = base
            idx_v[pl.ds(_NPOS_PAD + i * 16, 16)] = base + _HW

        _pos_streams = (
            (loc_hbm, 0, 0, 128), (loc_hbm, 128, 128, 80),
            (loc_hbm, 208, 208, 128), (loc_hbm, 336, 336, 80),
            (clf_hbm, 208, 416, 128), (clf_hbm, 336, 544, 80),
        )
        copies = [pltpu.async_copy(tab.at[idx_v.at[pl.ds(so, n)]],
                                   val_v.at[pl.ds(do, n)], sem)
                  for tab, so, do, n in _pos_streams]

        for i in range(_NBG // 16):
            q = lanes + (i * 16)
            bx = plsc.load_gather(pk_v, [_OFF_CT + q * 3 + 1])
            by = plsc.load_gather(pk_v, [_OFF_CT + q * 3 + 2])
            b = (jnp.where(q >= _NNEG, 1, 0)
                 + jnp.where(q >= 2 * _NNEG, 1, 0)
                 + jnp.where(q >= 3 * _NNEG, 1, 0))
            idx_v[pl.ds(2 * _NPOS_PAD + i * 16, 16)] = (
                b * _PSTR + by * _W + bx)

        _bg_streams = ((416, 624, 128), (544, 752, 128),
                       (672, 880, 128), (800, 1008, 16))
        copies += [pltpu.async_copy(clf_hbm.at[idx_v.at[pl.ds(so, n)]],
                                    val_v.at[pl.ds(do, n)], sem)
                   for so, do, n in _bg_streams]

        inv_da = plsc.bitcast(pk_v[pl.ds(_OFF_INV, 16)], jnp.float32)

        for c in copies:
            c.wait()

        sl_acc = jnp.zeros((16,), jnp.float32)
        car_acc = jnp.zeros((16,), jnp.float32)
        for i in range(_NPOS_PAD // 16):
            p = lanes + (i * 16)
            valid = p < _NPOS
            w = jnp.where(valid, 1.0, 0.0)
            psafe = jnp.where(valid, p, 0)
            g0 = plsc.bitcast(
                plsc.load_gather(pk_v, [_OFF_GT + psafe * 4]), jnp.float32)
            g1 = plsc.bitcast(
                plsc.load_gather(pk_v, [_OFF_GT + psafe * 4 + 1]), jnp.float32)
            g2 = plsc.bitcast(
                plsc.load_gather(pk_v, [_OFF_GT + psafe * 4 + 2]), jnp.float32)
            g3 = plsc.bitcast(
                plsc.load_gather(pk_v, [_OFF_GT + psafe * 4 + 3]), jnp.float32)
            x_gt = g0 + (g2 - g0) * 0.5
            y_gt = g1 - (g3 - g1) * 0.5
            dx = (x_gt - val_v[pl.ds(i * 16, 16)]) * inv_da
            dy = (y_gt - val_v[pl.ds(_NPOS_PAD + i * 16, 16)]) * inv_da
            sl_acc = sl_acc + w * (_huber(dx) + _huber(dy))
            car_acc = car_acc + w * _focal(
                val_v[pl.ds(2 * _NPOS_PAD + i * 16, 16)])

        bg_acc = jnp.zeros((16,), jnp.float32)
        for i in range(_NBG // 16):
            bg_acc = bg_acc + _focal(val_v[pl.ds(3 * _NPOS_PAD + i * 16, 16)])

        tot = (sl_acc * (_BETA_LOC / _NPOS)
               + car_acc * (1.0 / ((_B - 1) * (_NBOX - 1)))
               + bg_acc * (1.0 / ((_B - 1) * (_NNEG - 1))))
        out_v[...] = jnp.zeros((16,), jnp.float32) + jnp.sum(tot)
        pltpu.sync_copy(out_v, out_hbm)


def kernel(regression_targets, classification_targets_dict, gt_boxes_tensor,
           loc, size, clf, occupancy, angle, heading, anchor):
    rt = regression_targets.reshape(-1).astype(jnp.int32)
    ct = classification_targets_dict.reshape(-1).astype(jnp.int32)
    gt = lax.bitcast_convert_type(
        gt_boxes_tensor.reshape(-1).astype(jnp.float32), jnp.int32)
    a0 = anchor[0].astype(jnp.float32)
    a1 = anchor[1].astype(jnp.float32)
    inv_da = 1.0 / jnp.sqrt(a0 * a0 + a1 * a1)
    inv = lax.bitcast_convert_type(
        jnp.broadcast_to(inv_da, (16,)), jnp.int32)
    packed = jnp.concatenate([rt, ct, gt, inv])
    out = _loss_kernel(packed, loc.reshape(-1), clf.reshape(-1))
    return out[0]


# speedup vs baseline: 1.4649x; 1.4649x over previous
import dataclasses
import functools

import jax
import jax.numpy as jnp
from jax import lax
from jax.experimental import pallas as pl
from jax.experimental.pallas import tpu as pltpu
from jax.experimental.pallas import tpu_sc as plsc

_B, _NBOX, _NNEG = 4, 50, 100
_H, _W = 248, 216
_TY, _TX = _H // 8, 2
_TILE = 1024
_PLANE_W = _TY * _TX * _TILE
_NPOS = _B * _NBOX
_NBG = _B * _NNEG
_NPOS_PAD = 208
_NBG_PAD = 400
_NIDX = 3 * _NPOS_PAD + _NBG
_OFF_XS = 0
_OFF_YS = 208
_OFF_BX = 416
_OFF_BY = 816
_OFF_XG = 1216
_OFF_YG = 1424
_OFF_INV = 1632
_NPACK = 1648
_ALPHA = 0.25
_BETA_LOC = 2.0
_LN2 = 0.6931471805599453

_N_TABLE = 2 * _B * 2 * _PLANE_W


def _repack_kernel(loc_ref, clf_ref, rt_ref, ct_ref, gt_ref, anchor_ref,
                   tab_ref, pk_ref):
    for a, ref in enumerate((loc_ref, clf_ref)):
        for b in range(_B):
            for ch in range(2):
                tab_ref[a, b, ch, :, 0] = ref[b, 0, ch, :, 0:128].reshape(
                    _TY, 8, 128)
                tab_ref[a, b, ch, :, 1, :, 0:_W - 128] = ref[
                    b, 0, ch, :, 128:_W].reshape(_TY, 8, _W - 128)

    pk_ref[...] = jnp.zeros((_NPACK,), jnp.int32)
    pk_ref[_OFF_XS:_OFF_XS + _NPOS] = rt_ref[:, :, 0].reshape(_NPOS)
    pk_ref[_OFF_YS:_OFF_YS + _NPOS] = rt_ref[:, :, 1].reshape(_NPOS)
    pk_ref[_OFF_BX:_OFF_BX + _NBG] = ct_ref[:, :, 1].reshape(_NBG)
    pk_ref[_OFF_BY:_OFF_BY + _NBG] = ct_ref[:, :, 2].reshape(_NBG)
    g0 = gt_ref[:, :, 0].reshape(_NPOS)
    g1 = gt_ref[:, :, 1].reshape(_NPOS)
    g2 = gt_ref[:, :, 2].reshape(_NPOS)
    g3 = gt_ref[:, :, 3].reshape(_NPOS)
    x_gt = g0 + (g2 - g0) * 0.5
    y_gt = g1 - (g3 - g1) * 0.5
    pk_ref[_OFF_XG:_OFF_XG + _NPOS] = lax.bitcast_convert_type(
        x_gt, jnp.int32)
    pk_ref[_OFF_YG:_OFF_YG + _NPOS] = lax.bitcast_convert_type(
        y_gt, jnp.int32)
    a0 = anchor_ref[0]
    a1 = anchor_ref[1]
    inv_da = lax.rsqrt(a0 * a0 + a1 * a1)
    pk_ref[_OFF_INV:_OFF_INV + 16] = lax.bitcast_convert_type(
        jnp.broadcast_to(inv_da, (16,)), jnp.int32)


_repack = pl.pallas_call(
    _repack_kernel,
    out_shape=(
        jax.ShapeDtypeStruct((2, _B, 2, _TY, _TX, 8, 128), jnp.float32),
        jax.ShapeDtypeStruct((_NPACK,), jnp.int32),
    ),
    grid=(1,),
    in_specs=[
        pl.BlockSpec((_B, 1, 2, _H, _W), lambda i: (0, 0, 0, 0, 0)),
        pl.BlockSpec((_B, 1, 2, _H, _W), lambda i: (0, 0, 0, 0, 0)),
        pl.BlockSpec((_B, _NBOX, 2), lambda i: (0, 0, 0)),
        pl.BlockSpec((_B, _NNEG, 3), lambda i: (0, 0, 0)),
        pl.BlockSpec((_B, _NBOX, 4), lambda i: (0, 0, 0)),
        pl.BlockSpec((2,), lambda i: (0,)),
    ],
    out_specs=(
        pl.BlockSpec((2, _B, 2, _TY, _TX, 8, 128),
                     lambda i: (0, 0, 0, 0, 0, 0, 0)),
        pl.BlockSpec((_NPACK,), lambda i: (0,)),
    ),
)


def _ln(p):
    bits = lax.bitcast_convert_type(p, jnp.int32)
    e = jnp.right_shift(bits, 23) - 127
    m = lax.bitcast_convert_type(
        jnp.bitwise_or(jnp.bitwise_and(bits, 0x007FFFFF), 0x3F800000),
        jnp.float32)
    t = (m - 1.0) / (m + 1.0)
    t2 = t * t
    ln_m = t * (2.0 + t2 * (2.0 / 3.0 + t2 * (2.0 / 5.0
                + t2 * (2.0 / 7.0 + t2 * (2.0 / 9.0)))))
    return e.astype(jnp.float32) * _LN2 + ln_m


def _focal(p):
    one_m = 1.0 - p
    return -_ln(p) * (_ALPHA * one_m * one_m)


def _huber(x):
    ax = jnp.abs(x)
    return jnp.where(ax < 1.0, 0.5 * x * x, ax - 0.5)


def _tile_word(b, y, x):
    plane = b * 2
    tile = (plane * _TY + jnp.right_shift(y, 3)) * _TX + jnp.right_shift(x, 7)
    return (tile * _TILE + jnp.left_shift(jnp.bitwise_and(y, 7), 7)
            + jnp.bitwise_and(x, 127))


_mesh = plsc.VectorSubcoreMesh(core_axis_name="c", subcore_axis_name="s")

_cp = pltpu.CompilerParams()
if "needs_layout_passes" in pltpu.CompilerParams.__dataclass_fields__:
    _cp = dataclasses.replace(_cp, needs_layout_passes=False)


@functools.partial(
    pl.kernel,
    out_type=jax.ShapeDtypeStruct((16,), jnp.float32),
    mesh=_mesh,
    compiler_params=_cp,
    scratch_types=[
        pltpu.VMEM((_NPACK,), jnp.int32),
        pltpu.VMEM((_NIDX,), jnp.int32),
        pltpu.VMEM((_NIDX,), jnp.float32),
        pltpu.VMEM((16,), jnp.float32),
        pltpu.SemaphoreType.DMA,
    ],
)
def _loss_kernel(tab_hbm, pk_hbm, out_hbm,
                 pk_v, idx_v, val_v, out_v, sem):
    cid = lax.axis_index("c")
    sid = lax.axis_index("s")

    @pl.when(jnp.logical_and(cid == 0, sid == 0))
    def _():
        pltpu.sync_copy(pk_hbm, pk_v)

        lanes = lax.iota(jnp.int32, 16)

        for i in range(_NPOS_PAD // 16):
            p = lanes + (i * 16)
            x = pk_v[pl.ds(_OFF_XS + i * 16, 16)]
            y = pk_v[pl.ds(_OFF_YS + i * 16, 16)]
            b = (jnp.where(p >= _NBOX, 1, 0)
                 + jnp.where(p >= 2 * _NBOX, 1, 0)
                 + jnp.where(p >= 3 * _NBOX, 1, 0))
            base = _tile_word(b, y, x)
            idx_v[pl.ds(i * 16, 16)] = base
            idx_v[pl.ds(_NPOS_PAD + i * 16, 16)] = base + _PLANE_W
            idx_v[pl.ds(2 * _NPOS_PAD + i * 16, 16)] = base + 9 * _PLANE_W

        copies = [pltpu.async_copy(tab_hbm.at[idx_v.at[pl.ds(off, 128)]],
                                   val_v.at[pl.ds(off, 128)], sem)
                  for off in range(0, 512, 128)]

        for i in range(_NBG_PAD // 16):
            q = lanes + (i * 16)
            bx = pk_v[pl.ds(_OFF_BX + i * 16, 16)]
            by = pk_v[pl.ds(_OFF_BY + i * 16, 16)]
            b = (jnp.where(q >= _NNEG, 1, 0)
                 + jnp.where(q >= 2 * _NNEG, 1, 0)
                 + jnp.where(q >= 3 * _NNEG, 1, 0))
            idx_v[pl.ds(3 * _NPOS_PAD + i * 16, 16)] = (
                _tile_word(b, by, bx) + 8 * _PLANE_W)

        copies += [pltpu.async_copy(tab_hbm.at[idx_v.at[pl.ds(off, 128)]],
                                    val_v.at[pl.ds(off, 128)], sem)
                   for off in range(512, _NIDX, 128)]

        inv_da = plsc.bitcast(pk_v[pl.ds(_OFF_INV, 16)], jnp.float32)

        for c in copies:
            c.wait()

        sl_acc = jnp.zeros((16,), jnp.float32)
        car_acc = jnp.zeros((16,), jnp.float32)
        for i in range(_NPOS_PAD // 16):
            p = lanes + (i * 16)
            w = jnp.where(p < _NPOS, 1.0, 0.0)
            x_gt = plsc.bitcast(pk_v[pl.ds(_OFF_XG + i * 16, 16)],
                                jnp.float32)
            y_gt = plsc.bitcast(pk_v[pl.ds(_OFF_YG + i * 16, 16)],
                                jnp.float32)
            dx = (x_gt - val_v[pl.ds(i * 16, 16)]) * inv_da
            dy = (y_gt - val_v[pl.ds(_NPOS_PAD + i * 16, 16)]) * inv_da
            sl_acc = sl_acc + w * (_huber(dx) + _huber(dy))
            car_acc = car_acc + w * _focal(
                val_v[pl.ds(2 * _NPOS_PAD + i * 16, 16)])

        bg_acc = jnp.zeros((16,), jnp.float32)
        for i in range(_NBG_PAD // 16):
            bg_acc = bg_acc + _focal(val_v[pl.ds(3 * _NPOS_PAD + i * 16, 16)])

        tot = (sl_acc * (_BETA_LOC / _NPOS)
               + car_acc * (1.0 / ((_B - 1) * (_NBOX - 1)))
               + bg_acc * (1.0 / ((_B - 1) * (_NNEG - 1))))
        out_v[...] = jnp.zeros((16,), jnp.float32) + jnp.sum(tot)
        pltpu.sync_copy(out_v, out_hbm)


def kernel(regression_targets, classification_targets_dict, gt_boxes_tensor,
           loc, size, clf, occupancy, angle, heading, anchor):
    rt = regression_targets.astype(jnp.int32)
    ct = classification_targets_dict.astype(jnp.int32)
    table, packed = _repack(loc, clf, rt, ct,
                            gt_boxes_tensor.astype(jnp.float32),
                            anchor.astype(jnp.float32))
    out = _loss_kernel(table.reshape(-1), packed)
    return out[0]
